# Initial kernel scaffold; baseline (speedup 1.0000x reference)
import jax
import jax.numpy as jnp
from jax.experimental import pallas as pl


def kernel(net_weights, flat_node2pin_start, flat_node2pin, pin2net_map, flat_net2pin):
    raise NotImplementedError("write your pallas kernel here")


# speedup vs baseline: 1506.2125x; 1506.2125x over previous
import functools

import jax
import jax.numpy as jnp
from jax import lax
from jax.experimental import pallas as pl
from jax.experimental.pallas import tpu as pltpu
from jax.experimental.pallas import tpu_sc as plsc

P = 3200000
NETS = 500000
N = 100000
NM = 90000
NW = 32
PPW = P // NW
CH = 2000
NCH = PPW // CH
NPWK = N // NW
NGR = 196
OUTW = 3136
SPAD = 100096
SBUF = 3144

_mesh = plsc.VectorSubcoreMesh(core_axis_name="c", subcore_axis_name="s")


def _phase_a_body(lo_ref, hi_ref, w_ref, v_ref):
    d = hi_ref[...] - lo_ref[...]
    denom = jnp.maximum(d - 1, 1).astype(jnp.float32)
    v_ref[...] = jnp.where(d > 1, jnp.maximum(w_ref[...], 1.0) / denom, 0.0)


@functools.partial(
    pl.kernel,
    mesh=_mesh,
    compiler_params=pltpu.CompilerParams(needs_layout_passes=False),
    out_type=[
        jax.ShapeDtypeStruct((P,), jnp.float32),
        jax.ShapeDtypeStruct((NW, 16), jnp.float32),
    ],
    scratch_types=[
        pltpu.VMEM((CH,), jnp.int32),
        pltpu.VMEM((CH,), jnp.int32),
        pltpu.VMEM((CH,), jnp.float32),
        pltpu.VMEM((CH,), jnp.float32),
        pltpu.VMEM((16,), jnp.float32),
        pltpu.SemaphoreType.DMA,
        pltpu.SemaphoreType.DMA,
    ],
)
def _phase_b(q_hbm, p2n_hbm, v_hbm, lsg_hbm, wt_hbm,
             q_v, n_v, c_v, ls_v, wt_v, sem1, sem2):
    wid = lax.axis_index("s") * 2 + lax.axis_index("c")
    e0 = wid * PPW

    def chunk(ch, carry):
        base = e0 + ch * CH
        pltpu.sync_copy(q_hbm.at[pl.ds(base, CH)], q_v)
        pltpu.async_copy(p2n_hbm.at[q_v], n_v, sem1).wait()
        pltpu.async_copy(v_hbm.at[n_v], c_v, sem2).wait()

        def row(r, cr):
            x = c_v[pl.ds(r * 16, 16)]
            incl = plsc.cumsum(x)
            ls_v[pl.ds(r * 16, 16)] = incl - x + cr
            return cr + jnp.sum(x)

        carry = lax.fori_loop(0, CH // 16, row, carry)
        pltpu.sync_copy(ls_v, lsg_hbm.at[pl.ds(base, CH)])
        return carry

    tot = lax.fori_loop(0, NCH, chunk, jnp.float32(0.0))
    wt_v[...] = jnp.zeros((16,), jnp.float32) + tot
    pltpu.sync_copy(wt_v, wt_hbm.at[wid])


@functools.partial(
    pl.kernel,
    mesh=_mesh,
    compiler_params=pltpu.CompilerParams(needs_layout_passes=False),
    out_type=jax.ShapeDtypeStruct((NW, OUTW), jnp.float32),
    scratch_types=[
        pltpu.VMEM((SBUF,), jnp.int32),
        pltpu.VMEM((SBUF,), jnp.float32),
        pltpu.VMEM((NW, 16), jnp.float32),
        pltpu.VMEM((32,), jnp.float32),
        pltpu.VMEM((OUTW,), jnp.float32),
        pltpu.SemaphoreType.DMA,
    ],
)
def _phase_c(s_hbm, lsg_hbm, wt_hbm, out_hbm,
             s_v, g_v, wt_v, pt_v, o_v, sem):
    wid = lax.axis_index("s") * 2 + lax.axis_index("c")
    node0 = wid * NPWK
    base_al = (node0 // 8) * 8
    off = node0 - base_al

    pltpu.sync_copy(s_hbm.at[pl.ds(base_al, SBUF)], s_v)
    pltpu.async_copy(lsg_hbm.at[s_v], g_v, sem).wait()
    pltpu.sync_copy(wt_hbm, wt_v)

    iota = lax.iota(jnp.int32, 16)
    zero = jnp.zeros((16,), jnp.int32)
    t0 = plsc.load_gather(wt_v, [iota, zero])
    t1 = plsc.load_gather(wt_v, [iota + 16, zero])
    pt_v[pl.ds(0, 16)] = plsc.cumsum(t0) - t0
    pt_v[pl.ds(16, 16)] = plsc.cumsum(t1) - t1 + jnp.sum(t0)

    def grp(g, _):
        k = off + g * 16 + iota
        b = plsc.load_gather(s_v, [k])
        gb = plsc.load_gather(g_v, [k])
        e = plsc.load_gather(s_v, [k + 1])
        ge = plsc.load_gather(g_v, [k + 1])
        pb = plsc.load_gather(pt_v, [b // PPW])
        pe = plsc.load_gather(pt_v, [e // PPW])
        lcl = g * 16 + iota
        mask = (lcl < NPWK) & (node0 + lcl < NM)
        o_v[pl.ds(g * 16, 16)] = jnp.where(mask, (ge - gb) + (pe - pb), 0.0)
        return 0

    lax.fori_loop(0, NGR, grp, 0)
    pltpu.sync_copy(o_v, out_hbm.at[wid])


def kernel(net_weights, flat_node2pin_start, flat_node2pin, pin2net_map, flat_net2pin):
    pad = 512000 - NETS
    lo2d = jnp.pad(flat_net2pin[:NETS], (0, pad)).reshape(1000, 512)
    hi2d = jnp.pad(flat_net2pin[1:], (0, pad)).reshape(1000, 512)
    w2d = jnp.pad(net_weights, (0, pad)).reshape(1000, 512)
    v = pl.pallas_call(
        _phase_a_body,
        out_shape=jax.ShapeDtypeStruct((1000, 512), jnp.float32),
    )(lo2d, hi2d, w2d)

    lsg, wt = _phase_b(flat_node2pin, pin2net_map, v.reshape(512000))

    s_pad = jnp.pad(flat_node2pin_start, (0, SPAD - (N + 1)))
    out2d = _phase_c(s_pad, lsg, wt)
    return out2d[:, :NPWK].reshape(N)


# speedup vs baseline: 1885.2397x; 1.2516x over previous
import functools

import jax
import jax.numpy as jnp
from jax import lax
from jax.experimental import pallas as pl
from jax.experimental.pallas import tpu as pltpu
from jax.experimental.pallas import tpu_sc as plsc

P = 3200000
NETS = 500000
N = 100000
NM = 90000
NW = 32
PPW = P // NW
CH = 10000
NCH = PPW // CH
NPWK = N // NW
NGR = 196
OUTW = 3136
SPAD = 100096
SBUF = 3144

_mesh = plsc.VectorSubcoreMesh(core_axis_name="c", subcore_axis_name="s")


def _phase_a_body(lo_ref, hi_ref, w_ref, v_ref):
    d = hi_ref[...] - lo_ref[...]
    denom = jnp.maximum(d - 1, 1).astype(jnp.float32)
    v_ref[...] = jnp.where(d > 1, jnp.maximum(w_ref[...], 1.0) / denom, 0.0)


@functools.partial(
    pl.kernel,
    mesh=_mesh,
    compiler_params=pltpu.CompilerParams(needs_layout_passes=False),
    out_type=[
        jax.ShapeDtypeStruct((P,), jnp.float32),
        jax.ShapeDtypeStruct((NW, 16), jnp.float32),
    ],
    scratch_types=[
        pltpu.VMEM((CH,), jnp.int32),
        pltpu.VMEM((CH,), jnp.int32),
        pltpu.VMEM((CH,), jnp.float32),
        pltpu.VMEM((CH,), jnp.float32),
        pltpu.VMEM((16,), jnp.float32),
        pltpu.SemaphoreType.DMA,
        pltpu.SemaphoreType.DMA,
    ],
)
def _phase_b(q_hbm, p2n_hbm, v_hbm, lsg_hbm, wt_hbm,
             q_v, n_v, c_v, ls_v, wt_v, sem1, sem2):
    wid = lax.axis_index("s") * 2 + lax.axis_index("c")
    e0 = wid * PPW

    def chunk(ch, carry):
        base = e0 + ch * CH
        pltpu.sync_copy(q_hbm.at[pl.ds(base, CH)], q_v)
        pltpu.async_copy(p2n_hbm.at[q_v], n_v, sem1).wait()
        pltpu.async_copy(v_hbm.at[n_v], c_v, sem2).wait()

        def row(r, cr):
            x = c_v[pl.ds(r * 16, 16)]
            incl = plsc.cumsum(x)
            ls_v[pl.ds(r * 16, 16)] = incl - x + cr
            return cr + jnp.sum(x)

        carry = lax.fori_loop(0, CH // 16, row, carry)
        pltpu.sync_copy(ls_v, lsg_hbm.at[pl.ds(base, CH)])
        return carry

    tot = lax.fori_loop(0, NCH, chunk, jnp.float32(0.0))
    wt_v[...] = jnp.zeros((16,), jnp.float32) + tot
    pltpu.sync_copy(wt_v, wt_hbm.at[wid])


@functools.partial(
    pl.kernel,
    mesh=_mesh,
    compiler_params=pltpu.CompilerParams(needs_layout_passes=False),
    out_type=jax.ShapeDtypeStruct((NW, OUTW), jnp.float32),
    scratch_types=[
        pltpu.VMEM((SBUF,), jnp.int32),
        pltpu.VMEM((SBUF,), jnp.float32),
        pltpu.VMEM((NW, 16), jnp.float32),
        pltpu.VMEM((32,), jnp.float32),
        pltpu.VMEM((OUTW,), jnp.float32),
        pltpu.SemaphoreType.DMA,
    ],
)
def _phase_c(s_hbm, lsg_hbm, wt_hbm, out_hbm,
             s_v, g_v, wt_v, pt_v, o_v, sem):
    wid = lax.axis_index("s") * 2 + lax.axis_index("c")
    node0 = wid * NPWK
    base_al = (node0 // 8) * 8
    off = node0 - base_al

    pltpu.sync_copy(s_hbm.at[pl.ds(base_al, SBUF)], s_v)
    pltpu.async_copy(lsg_hbm.at[s_v], g_v, sem).wait()
    pltpu.sync_copy(wt_hbm, wt_v)

    iota = lax.iota(jnp.int32, 16)
    zero = jnp.zeros((16,), jnp.int32)
    t0 = plsc.load_gather(wt_v, [iota, zero])
    t1 = plsc.load_gather(wt_v, [iota + 16, zero])
    pt_v[pl.ds(0, 16)] = plsc.cumsum(t0) - t0
    pt_v[pl.ds(16, 16)] = plsc.cumsum(t1) - t1 + jnp.sum(t0)

    def grp(g, _):
        k = off + g * 16 + iota
        b = plsc.load_gather(s_v, [k])
        gb = plsc.load_gather(g_v, [k])
        e = plsc.load_gather(s_v, [k + 1])
        ge = plsc.load_gather(g_v, [k + 1])
        pb = plsc.load_gather(pt_v, [b // PPW])
        pe = plsc.load_gather(pt_v, [e // PPW])
        lcl = g * 16 + iota
        mask = (lcl < NPWK) & (node0 + lcl < NM)
        o_v[pl.ds(g * 16, 16)] = jnp.where(mask, (ge - gb) + (pe - pb), 0.0)
        return 0

    lax.fori_loop(0, NGR, grp, 0)
    pltpu.sync_copy(o_v, out_hbm.at[wid])


def kernel(net_weights, flat_node2pin_start, flat_node2pin, pin2net_map, flat_net2pin):
    pad = 512000 - NETS
    lo2d = jnp.pad(flat_net2pin[:NETS], (0, pad)).reshape(1000, 512)
    hi2d = jnp.pad(flat_net2pin[1:], (0, pad)).reshape(1000, 512)
    w2d = jnp.pad(net_weights, (0, pad)).reshape(1000, 512)
    v = pl.pallas_call(
        _phase_a_body,
        out_shape=jax.ShapeDtypeStruct((1000, 512), jnp.float32),
    )(lo2d, hi2d, w2d)

    lsg, wt = _phase_b(flat_node2pin, pin2net_map, v.reshape(512000))

    s_pad = jnp.pad(flat_node2pin_start, (0, SPAD - (N + 1)))
    out2d = _phase_c(s_pad, lsg, wt)
    return out2d[:, :NPWK].reshape(N)


# speedup vs baseline: 2537.3299x; 1.3459x over previous
import functools

import jax
import jax.numpy as jnp
from jax import lax
from jax.experimental import pallas as pl
from jax.experimental.pallas import tpu as pltpu
from jax.experimental.pallas import tpu_sc as plsc

P = 3200000
NETS = 500000
N = 100000
NM = 90000
NW = 32
PPW = P // NW
CH = 10000
NCH = PPW // CH
NPWK = N // NW
NGR = 196
OUTW = 3136
SPAD = 100096
SBUF = 3144

_mesh = plsc.VectorSubcoreMesh(core_axis_name="c", subcore_axis_name="s")


def _phase_a_body(lo_ref, hi_ref, w_ref, v_ref):
    d = hi_ref[...] - lo_ref[...]
    denom = jnp.maximum(d - 1, 1).astype(jnp.float32)
    v_ref[...] = jnp.where(d > 1, jnp.maximum(w_ref[...], 1.0) / denom, 0.0)


@functools.partial(
    pl.kernel,
    mesh=_mesh,
    compiler_params=pltpu.CompilerParams(needs_layout_passes=False),
    out_type=[
        jax.ShapeDtypeStruct((P,), jnp.float32),
        jax.ShapeDtypeStruct((NW, 16), jnp.float32),
    ],
    scratch_types=[
        pltpu.VMEM((CH,), jnp.int32),
        pltpu.VMEM((CH,), jnp.int32),
        pltpu.VMEM((CH,), jnp.float32),
        pltpu.VMEM((CH,), jnp.float32),
        pltpu.VMEM((16,), jnp.float32),
        pltpu.VMEM_SHARED((512000,), jnp.float32),
        pltpu.SemaphoreType.DMA,
        pltpu.SemaphoreType.DMA,
    ],
)
def _phase_b(q_hbm, p2n_hbm, v_hbm, lsg_hbm, wt_hbm,
             q_v, n_v, c_v, ls_v, wt_v, vs_sh, sem1, sem2):
    wid = lax.axis_index("s") * 2 + lax.axis_index("c")
    e0 = wid * PPW
    sid = lax.axis_index("s")
    stripe = 512000 // 16
    pltpu.sync_copy(v_hbm.at[pl.ds(sid * stripe, stripe)],
                    vs_sh.at[pl.ds(sid * stripe, stripe)])
    plsc.subcore_barrier()

    def chunk(ch, carry):
        base = e0 + ch * CH
        pltpu.sync_copy(q_hbm.at[pl.ds(base, CH)], q_v)
        pltpu.async_copy(p2n_hbm.at[q_v], n_v, sem1).wait()
        pltpu.async_copy(vs_sh.at[n_v], c_v, sem2).wait()

        def row(r, cr):
            x = c_v[pl.ds(r * 16, 16)]
            incl = plsc.cumsum(x)
            ls_v[pl.ds(r * 16, 16)] = incl - x + cr
            return cr + jnp.sum(x)

        carry = lax.fori_loop(0, CH // 16, row, carry)
        pltpu.sync_copy(ls_v, lsg_hbm.at[pl.ds(base, CH)])
        return carry

    tot = lax.fori_loop(0, NCH, chunk, jnp.float32(0.0))
    wt_v[...] = jnp.zeros((16,), jnp.float32) + tot
    pltpu.sync_copy(wt_v, wt_hbm.at[wid])


@functools.partial(
    pl.kernel,
    mesh=_mesh,
    compiler_params=pltpu.CompilerParams(needs_layout_passes=False),
    out_type=jax.ShapeDtypeStruct((NW, OUTW), jnp.float32),
    scratch_types=[
        pltpu.VMEM((SBUF,), jnp.int32),
        pltpu.VMEM((SBUF,), jnp.float32),
        pltpu.VMEM((NW, 16), jnp.float32),
        pltpu.VMEM((32,), jnp.float32),
        pltpu.VMEM((OUTW,), jnp.float32),
        pltpu.SemaphoreType.DMA,
    ],
)
def _phase_c(s_hbm, lsg_hbm, wt_hbm, out_hbm,
             s_v, g_v, wt_v, pt_v, o_v, sem):
    wid = lax.axis_index("s") * 2 + lax.axis_index("c")
    node0 = wid * NPWK
    base_al = (node0 // 8) * 8
    off = node0 - base_al

    pltpu.sync_copy(s_hbm.at[pl.ds(base_al, SBUF)], s_v)
    pltpu.async_copy(lsg_hbm.at[s_v], g_v, sem).wait()
    pltpu.sync_copy(wt_hbm, wt_v)

    iota = lax.iota(jnp.int32, 16)
    zero = jnp.zeros((16,), jnp.int32)
    t0 = plsc.load_gather(wt_v, [iota, zero])
    t1 = plsc.load_gather(wt_v, [iota + 16, zero])
    pt_v[pl.ds(0, 16)] = plsc.cumsum(t0) - t0
    pt_v[pl.ds(16, 16)] = plsc.cumsum(t1) - t1 + jnp.sum(t0)

    def grp(g, _):
        k = off + g * 16 + iota
        b = plsc.load_gather(s_v, [k])
        gb = plsc.load_gather(g_v, [k])
        e = plsc.load_gather(s_v, [k + 1])
        ge = plsc.load_gather(g_v, [k + 1])
        pb = plsc.load_gather(pt_v, [b // PPW])
        pe = plsc.load_gather(pt_v, [e // PPW])
        lcl = g * 16 + iota
        mask = (lcl < NPWK) & (node0 + lcl < NM)
        o_v[pl.ds(g * 16, 16)] = jnp.where(mask, (ge - gb) + (pe - pb), 0.0)
        return 0

    lax.fori_loop(0, NGR, grp, 0)
    pltpu.sync_copy(o_v, out_hbm.at[wid])


def kernel(net_weights, flat_node2pin_start, flat_node2pin, pin2net_map, flat_net2pin):
    pad = 512000 - NETS
    lo2d = jnp.pad(flat_net2pin[:NETS], (0, pad)).reshape(1000, 512)
    hi2d = jnp.pad(flat_net2pin[1:], (0, pad)).reshape(1000, 512)
    w2d = jnp.pad(net_weights, (0, pad)).reshape(1000, 512)
    v = pl.pallas_call(
        _phase_a_body,
        out_shape=jax.ShapeDtypeStruct((1000, 512), jnp.float32),
    )(lo2d, hi2d, w2d)

    lsg, wt = _phase_b(flat_node2pin, pin2net_map, v.reshape(512000))

    s_pad = jnp.pad(flat_node2pin_start, (0, SPAD - (N + 1)))
    out2d = _phase_c(s_pad, lsg, wt)
    return out2d[:, :NPWK].reshape(N)


# speedup vs baseline: 3285.4192x; 1.2948x over previous
import functools

import jax
import jax.numpy as jnp
from jax import lax
from jax.experimental import pallas as pl
from jax.experimental.pallas import tpu as pltpu
from jax.experimental.pallas import tpu_sc as plsc

P = 3200000
NETS = 500000
N = 100000
NM = 90000
NW = 32
PPW = P // NW
CH = 10000
NCH = PPW // CH
NPWK = N // NW
NGR = 196
OUTW = 3136
SPAD = 100096
SBUF = 3144

_mesh = plsc.VectorSubcoreMesh(core_axis_name="c", subcore_axis_name="s")


def _phase_a_body(lo_ref, hi_ref, w_ref, v_ref):
    d = hi_ref[...] - lo_ref[...]
    denom = jnp.maximum(d - 1, 1).astype(jnp.float32)
    v_ref[...] = jnp.where(d > 1, jnp.maximum(w_ref[...], 1.0) / denom, 0.0)


@functools.partial(
    pl.kernel,
    mesh=_mesh,
    compiler_params=pltpu.CompilerParams(needs_layout_passes=False),
    out_type=[
        jax.ShapeDtypeStruct((P,), jnp.float32),
        jax.ShapeDtypeStruct((NW, 16), jnp.float32),
    ],
    scratch_types=[
        pltpu.VMEM((CH,), jnp.int32),
        pltpu.VMEM((CH,), jnp.int32),
        pltpu.VMEM((CH,), jnp.int32),
        pltpu.VMEM((CH,), jnp.int32),
        pltpu.VMEM((CH,), jnp.float32),
        pltpu.VMEM((CH,), jnp.float32),
        pltpu.VMEM((CH,), jnp.float32),
        pltpu.VMEM((CH,), jnp.float32),
        pltpu.VMEM((16,), jnp.float32),
        pltpu.VMEM_SHARED((512000,), jnp.float32),
        pltpu.SemaphoreType.DMA,
        pltpu.SemaphoreType.DMA,
        pltpu.SemaphoreType.DMA,
        pltpu.SemaphoreType.DMA,
        pltpu.SemaphoreType.DMA,
        pltpu.SemaphoreType.DMA,
        pltpu.SemaphoreType.DMA,
        pltpu.SemaphoreType.DMA,
    ],
)
def _phase_b(q_hbm, p2n_hbm, v_hbm, lsg_hbm, wt_hbm,
             q0_v, q1_v, n0_v, n1_v, c0_v, c1_v, ls0_v, ls1_v, wt_v, vs_sh,
             sq0, sq1, s1a, s1b, s2a, s2b, sw0, sw1):
    wid = lax.axis_index("s") * 2 + lax.axis_index("c")
    e0 = wid * PPW
    sid = lax.axis_index("s")
    stripe = 512000 // 16
    pltpu.sync_copy(v_hbm.at[pl.ds(sid * stripe, stripe)],
                    vs_sh.at[pl.ds(sid * stripe, stripe)])
    plsc.subcore_barrier()

    def prefix(c_v, ls_v, carry):
        def row(r, cr):
            x = c_v[pl.ds(r * 16, 16)]
            incl = plsc.cumsum(x)
            ls_v[pl.ds(r * 16, 16)] = incl - x + cr
            return cr + jnp.sum(x)

        return lax.fori_loop(0, CH // 16, row, carry)

    npair = NCH // 2
    pltpu.async_copy(q_hbm.at[pl.ds(e0, CH)], q0_v, sq0)
    pltpu.async_copy(q_hbm.at[pl.ds(e0 + CH, CH)], q1_v, sq1)
    pltpu.make_async_copy(q_hbm.at[pl.ds(0, CH)], q0_v, sq0).wait()
    pltpu.async_copy(p2n_hbm.at[q0_v], n0_v, s1a)

    def pair(ip, carry):
        base0 = e0 + (2 * ip) * CH
        base1 = base0 + CH
        pltpu.make_async_copy(p2n_hbm.at[q0_v], n0_v, s1a).wait()
        pltpu.async_copy(vs_sh.at[n0_v], c0_v, s2a)
        pltpu.make_async_copy(q_hbm.at[pl.ds(0, CH)], q1_v, sq1).wait()
        pltpu.async_copy(p2n_hbm.at[q1_v], n1_v, s1b)

        @pl.when(ip < npair - 1)
        def _():
            pltpu.async_copy(q_hbm.at[pl.ds(base0 + 2 * CH, CH)], q0_v, sq0)

        pltpu.make_async_copy(vs_sh.at[n0_v], c0_v, s2a).wait()

        @pl.when(ip >= 1)
        def _():
            pltpu.make_async_copy(ls0_v, lsg_hbm.at[pl.ds(0, CH)], sw0).wait()

        carry = prefix(c0_v, ls0_v, carry)
        pltpu.async_copy(ls0_v, lsg_hbm.at[pl.ds(base0, CH)], sw0)

        pltpu.make_async_copy(p2n_hbm.at[q1_v], n1_v, s1b).wait()
        pltpu.async_copy(vs_sh.at[n1_v], c1_v, s2b)

        @pl.when(ip < npair - 1)
        def _():
            pltpu.async_copy(q_hbm.at[pl.ds(base1 + 2 * CH, CH)], q1_v, sq1)

        @pl.when(ip < npair - 1)
        def _():
            pltpu.make_async_copy(q_hbm.at[pl.ds(0, CH)], q0_v, sq0).wait()
            pltpu.async_copy(p2n_hbm.at[q0_v], n0_v, s1a)

        pltpu.make_async_copy(vs_sh.at[n1_v], c1_v, s2b).wait()

        @pl.when(ip >= 1)
        def _():
            pltpu.make_async_copy(ls1_v, lsg_hbm.at[pl.ds(0, CH)], sw1).wait()

        carry = prefix(c1_v, ls1_v, carry)
        pltpu.async_copy(ls1_v, lsg_hbm.at[pl.ds(base1, CH)], sw1)
        return carry

    tot = lax.fori_loop(0, npair, pair, jnp.float32(0.0))
    pltpu.make_async_copy(ls0_v, lsg_hbm.at[pl.ds(0, CH)], sw0).wait()
    pltpu.make_async_copy(ls1_v, lsg_hbm.at[pl.ds(0, CH)], sw1).wait()
    wt_v[...] = jnp.zeros((16,), jnp.float32) + tot
    pltpu.sync_copy(wt_v, wt_hbm.at[wid])


@functools.partial(
    pl.kernel,
    mesh=_mesh,
    compiler_params=pltpu.CompilerParams(needs_layout_passes=False),
    out_type=jax.ShapeDtypeStruct((NW, OUTW), jnp.float32),
    scratch_types=[
        pltpu.VMEM((SBUF,), jnp.int32),
        pltpu.VMEM((SBUF,), jnp.float32),
        pltpu.VMEM((NW, 16), jnp.float32),
        pltpu.VMEM((32,), jnp.float32),
        pltpu.VMEM((OUTW,), jnp.float32),
        pltpu.SemaphoreType.DMA,
    ],
)
def _phase_c(s_hbm, lsg_hbm, wt_hbm, out_hbm,
             s_v, g_v, wt_v, pt_v, o_v, sem):
    wid = lax.axis_index("s") * 2 + lax.axis_index("c")
    node0 = wid * NPWK
    base_al = (node0 // 8) * 8
    off = node0 - base_al

    pltpu.sync_copy(s_hbm.at[pl.ds(base_al, SBUF)], s_v)
    pltpu.async_copy(lsg_hbm.at[s_v], g_v, sem).wait()
    pltpu.sync_copy(wt_hbm, wt_v)

    iota = lax.iota(jnp.int32, 16)
    zero = jnp.zeros((16,), jnp.int32)
    t0 = plsc.load_gather(wt_v, [iota, zero])
    t1 = plsc.load_gather(wt_v, [iota + 16, zero])
    pt_v[pl.ds(0, 16)] = plsc.cumsum(t0) - t0
    pt_v[pl.ds(16, 16)] = plsc.cumsum(t1) - t1 + jnp.sum(t0)

    def grp(g, _):
        k = off + g * 16 + iota
        b = plsc.load_gather(s_v, [k])
        gb = plsc.load_gather(g_v, [k])
        e = plsc.load_gather(s_v, [k + 1])
        ge = plsc.load_gather(g_v, [k + 1])
        pb = plsc.load_gather(pt_v, [b // PPW])
        pe = plsc.load_gather(pt_v, [e // PPW])
        lcl = g * 16 + iota
        mask = (lcl < NPWK) & (node0 + lcl < NM)
        o_v[pl.ds(g * 16, 16)] = jnp.where(mask, (ge - gb) + (pe - pb), 0.0)
        return 0

    lax.fori_loop(0, NGR, grp, 0)
    pltpu.sync_copy(o_v, out_hbm.at[wid])


def kernel(net_weights, flat_node2pin_start, flat_node2pin, pin2net_map, flat_net2pin):
    pad = 512000 - NETS
    lo2d = jnp.pad(flat_net2pin[:NETS], (0, pad)).reshape(1000, 512)
    hi2d = jnp.pad(flat_net2pin[1:], (0, pad)).reshape(1000, 512)
    w2d = jnp.pad(net_weights, (0, pad)).reshape(1000, 512)
    v = pl.pallas_call(
        _phase_a_body,
        out_shape=jax.ShapeDtypeStruct((1000, 512), jnp.float32),
    )(lo2d, hi2d, w2d)

    lsg, wt = _phase_b(flat_node2pin, pin2net_map, v.reshape(512000))

    s_pad = jnp.pad(flat_node2pin_start, (0, SPAD - (N + 1)))
    out2d = _phase_c(s_pad, lsg, wt)
    return out2d[:, :NPWK].reshape(N)


# speedup vs baseline: 3287.2612x; 1.0006x over previous
import functools

import jax
import jax.numpy as jnp
from jax import lax
from jax.experimental import pallas as pl
from jax.experimental.pallas import tpu as pltpu
from jax.experimental.pallas import tpu_sc as plsc

P = 3200000
NETS = 500000
N = 100000
NM = 90000
NW = 32
PPW = P // NW
CH = 10000
NCH = PPW // CH
NPWK = N // NW
NGR = 196
OUTW = 3136
SPAD = 100096
SBUF = 3144

_mesh = plsc.VectorSubcoreMesh(core_axis_name="c", subcore_axis_name="s")


def _phase_a_body(lo_ref, hi_ref, w_ref, v_ref):
    d = hi_ref[...] - lo_ref[...]
    denom = jnp.maximum(d - 1, 1).astype(jnp.float32)
    v_ref[...] = jnp.where(d > 1, jnp.maximum(w_ref[...], 1.0) / denom, 0.0)


@functools.partial(
    pl.kernel,
    mesh=_mesh,
    compiler_params=pltpu.CompilerParams(needs_layout_passes=False),
    out_type=[
        jax.ShapeDtypeStruct((P,), jnp.float32),
        jax.ShapeDtypeStruct((NW, 16), jnp.float32),
    ],
    scratch_types=[
        pltpu.VMEM((CH,), jnp.int32),
        pltpu.VMEM((CH,), jnp.int32),
        pltpu.VMEM((CH,), jnp.int32),
        pltpu.VMEM((CH,), jnp.int32),
        pltpu.VMEM((CH,), jnp.float32),
        pltpu.VMEM((CH,), jnp.float32),
        pltpu.VMEM((CH,), jnp.float32),
        pltpu.VMEM((CH,), jnp.float32),
        pltpu.VMEM((16,), jnp.float32),
        pltpu.VMEM_SHARED((512000,), jnp.float32),
        pltpu.SemaphoreType.DMA,
        pltpu.SemaphoreType.DMA,
        pltpu.SemaphoreType.DMA,
        pltpu.SemaphoreType.DMA,
        pltpu.SemaphoreType.DMA,
        pltpu.SemaphoreType.DMA,
        pltpu.SemaphoreType.DMA,
        pltpu.SemaphoreType.DMA,
    ],
)
def _phase_b(q_hbm, p2n_hbm, v_hbm, lsg_hbm, wt_hbm,
             q0_v, q1_v, n0_v, n1_v, c0_v, c1_v, ls0_v, ls1_v, wt_v, vs_sh,
             sq0, sq1, s1a, s1b, s2a, s2b, sw0, sw1):
    wid = lax.axis_index("s") * 2 + lax.axis_index("c")
    e0 = wid * PPW
    sid = lax.axis_index("s")
    stripe = 512000 // 16
    pltpu.sync_copy(v_hbm.at[pl.ds(sid * stripe, stripe)],
                    vs_sh.at[pl.ds(sid * stripe, stripe)])
    plsc.subcore_barrier()

    def prefix(c_v, ls_v, carry):
        U = 5

        def row(r, cr):
            xs = [c_v[pl.ds((r * U + u) * 16, 16)] for u in range(U)]
            incls = [plsc.cumsum(x) for x in xs]
            for u in range(U):
                ls_v[pl.ds((r * U + u) * 16, 16)] = incls[u] - xs[u] + cr
                cr = cr + incls[u][15]
            return cr

        return lax.fori_loop(0, CH // 16 // U, row, carry)

    npair = NCH // 2
    pltpu.async_copy(q_hbm.at[pl.ds(e0, CH)], q0_v, sq0)
    pltpu.async_copy(q_hbm.at[pl.ds(e0 + CH, CH)], q1_v, sq1)
    pltpu.make_async_copy(q_hbm.at[pl.ds(0, CH)], q0_v, sq0).wait()
    pltpu.async_copy(p2n_hbm.at[q0_v], n0_v, s1a)

    def pair(ip, carry):
        base0 = e0 + (2 * ip) * CH
        base1 = base0 + CH
        pltpu.make_async_copy(p2n_hbm.at[q0_v], n0_v, s1a).wait()
        pltpu.async_copy(vs_sh.at[n0_v], c0_v, s2a)
        pltpu.make_async_copy(q_hbm.at[pl.ds(0, CH)], q1_v, sq1).wait()
        pltpu.async_copy(p2n_hbm.at[q1_v], n1_v, s1b)

        @pl.when(ip < npair - 1)
        def _():
            pltpu.async_copy(q_hbm.at[pl.ds(base0 + 2 * CH, CH)], q0_v, sq0)

        pltpu.make_async_copy(vs_sh.at[n0_v], c0_v, s2a).wait()

        @pl.when(ip >= 1)
        def _():
            pltpu.make_async_copy(ls0_v, lsg_hbm.at[pl.ds(0, CH)], sw0).wait()

        carry = prefix(c0_v, ls0_v, carry)
        pltpu.async_copy(ls0_v, lsg_hbm.at[pl.ds(base0, CH)], sw0)

        pltpu.make_async_copy(p2n_hbm.at[q1_v], n1_v, s1b).wait()
        pltpu.async_copy(vs_sh.at[n1_v], c1_v, s2b)

        @pl.when(ip < npair - 1)
        def _():
            pltpu.async_copy(q_hbm.at[pl.ds(base1 + 2 * CH, CH)], q1_v, sq1)

        @pl.when(ip < npair - 1)
        def _():
            pltpu.make_async_copy(q_hbm.at[pl.ds(0, CH)], q0_v, sq0).wait()
            pltpu.async_copy(p2n_hbm.at[q0_v], n0_v, s1a)

        pltpu.make_async_copy(vs_sh.at[n1_v], c1_v, s2b).wait()

        @pl.when(ip >= 1)
        def _():
            pltpu.make_async_copy(ls1_v, lsg_hbm.at[pl.ds(0, CH)], sw1).wait()

        carry = prefix(c1_v, ls1_v, carry)
        pltpu.async_copy(ls1_v, lsg_hbm.at[pl.ds(base1, CH)], sw1)
        return carry

    tot = lax.fori_loop(0, npair, pair, jnp.float32(0.0))
    pltpu.make_async_copy(ls0_v, lsg_hbm.at[pl.ds(0, CH)], sw0).wait()
    pltpu.make_async_copy(ls1_v, lsg_hbm.at[pl.ds(0, CH)], sw1).wait()
    wt_v[...] = jnp.zeros((16,), jnp.float32) + tot
    pltpu.sync_copy(wt_v, wt_hbm.at[wid])


@functools.partial(
    pl.kernel,
    mesh=_mesh,
    compiler_params=pltpu.CompilerParams(needs_layout_passes=False),
    out_type=jax.ShapeDtypeStruct((NW, OUTW), jnp.float32),
    scratch_types=[
        pltpu.VMEM((SBUF,), jnp.int32),
        pltpu.VMEM((SBUF,), jnp.float32),
        pltpu.VMEM((NW, 16), jnp.float32),
        pltpu.VMEM((32,), jnp.float32),
        pltpu.VMEM((OUTW,), jnp.float32),
        pltpu.SemaphoreType.DMA,
    ],
)
def _phase_c(s_hbm, lsg_hbm, wt_hbm, out_hbm,
             s_v, g_v, wt_v, pt_v, o_v, sem):
    wid = lax.axis_index("s") * 2 + lax.axis_index("c")
    node0 = wid * NPWK
    base_al = (node0 // 8) * 8
    off = node0 - base_al

    pltpu.sync_copy(s_hbm.at[pl.ds(base_al, SBUF)], s_v)
    pltpu.async_copy(lsg_hbm.at[s_v], g_v, sem).wait()
    pltpu.sync_copy(wt_hbm, wt_v)

    iota = lax.iota(jnp.int32, 16)
    zero = jnp.zeros((16,), jnp.int32)
    t0 = plsc.load_gather(wt_v, [iota, zero])
    t1 = plsc.load_gather(wt_v, [iota + 16, zero])
    pt_v[pl.ds(0, 16)] = plsc.cumsum(t0) - t0
    pt_v[pl.ds(16, 16)] = plsc.cumsum(t1) - t1 + jnp.sum(t0)

    def grp(g4, _):
        for u in range(4):
            g = g4 * 4 + u
            k = off + g * 16 + iota
            b = plsc.load_gather(s_v, [k])
            gb = plsc.load_gather(g_v, [k])
            e = plsc.load_gather(s_v, [k + 1])
            ge = plsc.load_gather(g_v, [k + 1])
            pb = plsc.load_gather(pt_v, [b // PPW])
            pe = plsc.load_gather(pt_v, [e // PPW])
            lcl = g * 16 + iota
            mask = (lcl < NPWK) & (node0 + lcl < NM)
            o_v[pl.ds(g * 16, 16)] = jnp.where(mask, (ge - gb) + (pe - pb), 0.0)
        return 0

    lax.fori_loop(0, NGR // 4, grp, 0)
    pltpu.sync_copy(o_v, out_hbm.at[wid])


def kernel(net_weights, flat_node2pin_start, flat_node2pin, pin2net_map, flat_net2pin):
    pad = 512000 - NETS
    lo2d = jnp.pad(flat_net2pin[:NETS], (0, pad)).reshape(1000, 512)
    hi2d = jnp.pad(flat_net2pin[1:], (0, pad)).reshape(1000, 512)
    w2d = jnp.pad(net_weights, (0, pad)).reshape(1000, 512)
    v = pl.pallas_call(
        _phase_a_body,
        out_shape=jax.ShapeDtypeStruct((1000, 512), jnp.float32),
    )(lo2d, hi2d, w2d)

    lsg, wt = _phase_b(flat_node2pin, pin2net_map, v.reshape(512000))

    s_pad = jnp.pad(flat_node2pin_start, (0, SPAD - (N + 1)))
    out2d = _phase_c(s_pad, lsg, wt)
    return out2d[:, :NPWK].reshape(N)


# speedup vs baseline: 3431.3588x; 1.0438x over previous
import functools

import jax
import jax.numpy as jnp
from jax import lax
from jax.experimental import pallas as pl
from jax.experimental.pallas import tpu as pltpu
from jax.experimental.pallas import tpu_sc as plsc

P = 3200000
NETS = 500000
N = 100000
NM = 90000
NW = 32
PPW = P // NW
CH = 10000
NCH = PPW // CH
NPWK = N // NW
NGR = 196
OUTW = 3136
SPAD = 100096
SBUF = 3144

_mesh = plsc.VectorSubcoreMesh(core_axis_name="c", subcore_axis_name="s")


@functools.partial(
    pl.kernel,
    mesh=_mesh,
    compiler_params=pltpu.CompilerParams(needs_layout_passes=False),
    out_type=[
        jax.ShapeDtypeStruct((P,), jnp.float32),
        jax.ShapeDtypeStruct((NW, 16), jnp.float32),
    ],
    scratch_types=[
        pltpu.VMEM((CH,), jnp.int32),
        pltpu.VMEM((CH,), jnp.int32),
        pltpu.VMEM((CH,), jnp.int32),
        pltpu.VMEM((CH,), jnp.int32),
        pltpu.VMEM((CH,), jnp.float32),
        pltpu.VMEM((CH,), jnp.float32),
        pltpu.VMEM((CH,), jnp.float32),
        pltpu.VMEM((CH,), jnp.float32),
        pltpu.VMEM((16,), jnp.float32),
        pltpu.VMEM_SHARED((512000,), jnp.float32),
        pltpu.SemaphoreType.DMA,
        pltpu.SemaphoreType.DMA,
        pltpu.SemaphoreType.DMA,
        pltpu.SemaphoreType.DMA,
        pltpu.SemaphoreType.DMA,
        pltpu.SemaphoreType.DMA,
        pltpu.SemaphoreType.DMA,
        pltpu.SemaphoreType.DMA,
    ],
)
def _phase_b(q_hbm, p2n_hbm, w_hbm, n2p_hbm, lsg_hbm, wt_hbm,
             q0_v, q1_v, n0_v, n1_v, c0_v, c1_v, ls0_v, ls1_v, wt_v, vs_sh,
             sq0, sq1, s1a, s1b, s2a, s2b, sw0, sw1):
    wid = lax.axis_index("s") * 2 + lax.axis_index("c")
    e0 = wid * PPW
    sid = lax.axis_index("s")

    def prefix(c_v, ls_v, carry):
        U = 5

        def row(r, cr):
            xs = [c_v[pl.ds((r * U + u) * 16, 16)] for u in range(U)]
            incls = [plsc.cumsum(x) for x in xs]
            for u in range(U):
                ls_v[pl.ds((r * U + u) * 16, 16)] = incls[u] - xs[u] + cr
                cr = cr + incls[u][15]
            return cr

        return lax.fori_loop(0, CH // 16 // U, row, carry)

    npair = NCH // 2
    pltpu.async_copy(q_hbm.at[pl.ds(e0, CH)], q0_v, sq0)
    pltpu.async_copy(q_hbm.at[pl.ds(e0 + CH, CH)], q1_v, sq1)
    pltpu.make_async_copy(q_hbm.at[pl.ds(0, CH)], q0_v, sq0).wait()
    pltpu.async_copy(p2n_hbm.at[q0_v], n0_v, s1a)

    for j in range(4):
        nb = sid * 32000 + j * 8000
        pltpu.sync_copy(n2p_hbm.at[pl.ds(nb, 8016)], n1_v.at[pl.ds(0, 8016)])
        pltpu.sync_copy(w_hbm.at[pl.ds(nb, 8000)], c1_v.at[pl.ds(0, 8000)])

        def vrow(r, _):
            for u in range(4):
                o = (r * 4 + u) * 16
                x_lo = n1_v[pl.ds(o, 16)]
                x_hi = n1_v[pl.ds(o + 1, 16)]
                d = x_hi - x_lo
                den = jnp.maximum(d - 1, 1).astype(jnp.float32)
                wv = c1_v[pl.ds(o, 16)]
                ls1_v[pl.ds(o, 16)] = jnp.where(
                    d > 1, jnp.maximum(wv, 1.0) / den, 0.0)
            return 0

        lax.fori_loop(0, 125, vrow, 0)
        pltpu.sync_copy(ls1_v.at[pl.ds(0, 8000)], vs_sh.at[pl.ds(nb, 8000)])
    plsc.subcore_barrier()

    def pair(ip, carry):
        base0 = e0 + (2 * ip) * CH
        base1 = base0 + CH
        pltpu.make_async_copy(p2n_hbm.at[q0_v], n0_v, s1a).wait()
        pltpu.async_copy(vs_sh.at[n0_v], c0_v, s2a)
        pltpu.make_async_copy(q_hbm.at[pl.ds(0, CH)], q1_v, sq1).wait()
        pltpu.async_copy(p2n_hbm.at[q1_v], n1_v, s1b)

        @pl.when(ip < npair - 1)
        def _():
            pltpu.async_copy(q_hbm.at[pl.ds(base0 + 2 * CH, CH)], q0_v, sq0)

        pltpu.make_async_copy(vs_sh.at[n0_v], c0_v, s2a).wait()

        @pl.when(ip >= 1)
        def _():
            pltpu.make_async_copy(ls0_v, lsg_hbm.at[pl.ds(0, CH)], sw0).wait()

        carry = prefix(c0_v, ls0_v, carry)
        pltpu.async_copy(ls0_v, lsg_hbm.at[pl.ds(base0, CH)], sw0)

        pltpu.make_async_copy(p2n_hbm.at[q1_v], n1_v, s1b).wait()
        pltpu.async_copy(vs_sh.at[n1_v], c1_v, s2b)

        @pl.when(ip < npair - 1)
        def _():
            pltpu.async_copy(q_hbm.at[pl.ds(base1 + 2 * CH, CH)], q1_v, sq1)

        @pl.when(ip < npair - 1)
        def _():
            pltpu.make_async_copy(q_hbm.at[pl.ds(0, CH)], q0_v, sq0).wait()
            pltpu.async_copy(p2n_hbm.at[q0_v], n0_v, s1a)

        pltpu.make_async_copy(vs_sh.at[n1_v], c1_v, s2b).wait()

        @pl.when(ip >= 1)
        def _():
            pltpu.make_async_copy(ls1_v, lsg_hbm.at[pl.ds(0, CH)], sw1).wait()

        carry = prefix(c1_v, ls1_v, carry)
        pltpu.async_copy(ls1_v, lsg_hbm.at[pl.ds(base1, CH)], sw1)
        return carry

    tot = lax.fori_loop(0, npair, pair, jnp.float32(0.0))
    pltpu.make_async_copy(ls0_v, lsg_hbm.at[pl.ds(0, CH)], sw0).wait()
    pltpu.make_async_copy(ls1_v, lsg_hbm.at[pl.ds(0, CH)], sw1).wait()
    wt_v[...] = jnp.zeros((16,), jnp.float32) + tot
    pltpu.sync_copy(wt_v, wt_hbm.at[wid])


@functools.partial(
    pl.kernel,
    mesh=_mesh,
    compiler_params=pltpu.CompilerParams(needs_layout_passes=False),
    out_type=jax.ShapeDtypeStruct((NW, OUTW), jnp.float32),
    scratch_types=[
        pltpu.VMEM((SBUF,), jnp.int32),
        pltpu.VMEM((SBUF,), jnp.float32),
        pltpu.VMEM((NW, 16), jnp.float32),
        pltpu.VMEM((32,), jnp.float32),
        pltpu.VMEM((OUTW,), jnp.float32),
        pltpu.SemaphoreType.DMA,
    ],
)
def _phase_c(s_hbm, lsg_hbm, wt_hbm, out_hbm,
             s_v, g_v, wt_v, pt_v, o_v, sem):
    wid = lax.axis_index("s") * 2 + lax.axis_index("c")
    node0 = wid * NPWK
    base_al = (node0 // 8) * 8
    off = node0 - base_al

    pltpu.sync_copy(s_hbm.at[pl.ds(base_al, SBUF)], s_v)
    pltpu.async_copy(lsg_hbm.at[s_v], g_v, sem).wait()
    pltpu.sync_copy(wt_hbm, wt_v)

    iota = lax.iota(jnp.int32, 16)
    zero = jnp.zeros((16,), jnp.int32)
    t0 = plsc.load_gather(wt_v, [iota, zero])
    t1 = plsc.load_gather(wt_v, [iota + 16, zero])
    pt_v[pl.ds(0, 16)] = plsc.cumsum(t0) - t0
    pt_v[pl.ds(16, 16)] = plsc.cumsum(t1) - t1 + jnp.sum(t0)

    def grp(g4, _):
        for u in range(4):
            g = g4 * 4 + u
            k = off + g * 16 + iota
            b = plsc.load_gather(s_v, [k])
            gb = plsc.load_gather(g_v, [k])
            e = plsc.load_gather(s_v, [k + 1])
            ge = plsc.load_gather(g_v, [k + 1])
            pb = plsc.load_gather(pt_v, [b // PPW])
            pe = plsc.load_gather(pt_v, [e // PPW])
            lcl = g * 16 + iota
            mask = (lcl < NPWK) & (node0 + lcl < NM)
            o_v[pl.ds(g * 16, 16)] = jnp.where(mask, (ge - gb) + (pe - pb), 0.0)
        return 0

    lax.fori_loop(0, NGR // 4, grp, 0)
    pltpu.sync_copy(o_v, out_hbm.at[wid])


def kernel(net_weights, flat_node2pin_start, flat_node2pin, pin2net_map, flat_net2pin):
    w_pad = jnp.pad(net_weights, (0, 512000 - NETS))
    n2p_pad = jnp.pad(flat_net2pin, (0, 512024 - (NETS + 1)))
    lsg, wt = _phase_b(flat_node2pin, pin2net_map, w_pad, n2p_pad)

    s_pad = jnp.pad(flat_node2pin_start, (0, SPAD - (N + 1)))
    out2d = _phase_c(s_pad, lsg, wt)
    return out2d[:, :NPWK].reshape(N)
